# baseline (speedup 1.0000x reference)
import functools

import jax
import jax.numpy as jnp
from jax import lax
from jax.experimental import pallas as pl
from jax.experimental.pallas import tpu as pltpu
from jax.experimental.pallas import tpu_sc as plsc

NUM_CORES = 2
NUM_SUBCORES = 16
LANES = 16
NUM_WORKERS = NUM_CORES * NUM_SUBCORES

BATCH = 16384
EMBED_DIM = 32
B_PER_W = BATCH // NUM_WORKERS
GATHER_CHUNK = 128
N_CHUNKS = B_PER_W // GATHER_CHUNK


def _sc_body(uidx_hbm, iidx_hbm, uemb_hbm, iemb_hbm, out_hbm,
             uidx_v, iidx_v, urows_v, irows_v, out_v, sem_u, sem_i):
    wid = lax.axis_index("s") * NUM_CORES + lax.axis_index("c")
    base = wid * B_PER_W

    pltpu.sync_copy(uidx_hbm.at[pl.ds(base, B_PER_W)], uidx_v)
    pltpu.sync_copy(iidx_hbm.at[pl.ds(base, B_PER_W)], iidx_v)

    copies = []
    for j in range(N_CHUNKS):
        sl = pl.ds(j * GATHER_CHUNK, GATHER_CHUNK)
        copies.append(pltpu.async_copy(
            uemb_hbm.at[uidx_v.at[sl]], urows_v.at[sl], sem_u))
        copies.append(pltpu.async_copy(
            iemb_hbm.at[iidx_v.at[sl]], irows_v.at[sl], sem_i))
    for c in copies:
        c.wait()

    def group(g, carry):
        rows = lax.iota(jnp.int32, LANES) + g * LANES
        acc = jnp.zeros((LANES,), jnp.float32)
        for d in range(EMBED_DIM):
            cols = jnp.full((LANES,), d, jnp.int32)
            u = plsc.load_gather(urows_v, [rows, cols])
            it = plsc.load_gather(irows_v, [rows, cols])
            acc = acc + u * it
        out_v[pl.ds(g * LANES, LANES)] = acc
        return carry

    lax.fori_loop(0, B_PER_W // LANES, group, 0, unroll=False)

    pltpu.sync_copy(out_v, out_hbm.at[pl.ds(base, B_PER_W)])


@functools.partial(jax.jit, static_argnames=())
def kernel(user_indices, item_indices, user_emb, item_emb):
    mesh = plsc.VectorSubcoreMesh(
        core_axis_name="c", subcore_axis_name="s",
        num_cores=NUM_CORES, num_subcores=NUM_SUBCORES)
    run = pl.kernel(
        _sc_body,
        out_type=jax.ShapeDtypeStruct((BATCH,), jnp.float32),
        mesh=mesh,
        compiler_params=pltpu.CompilerParams(
            needs_layout_passes=False, use_tc_tiling_on_sc=False),
        scratch_types=[
            pltpu.VMEM((B_PER_W,), jnp.int32),
            pltpu.VMEM((B_PER_W,), jnp.int32),
            pltpu.VMEM((B_PER_W, EMBED_DIM), jnp.float32),
            pltpu.VMEM((B_PER_W, EMBED_DIM), jnp.float32),
            pltpu.VMEM((B_PER_W,), jnp.float32),
            pltpu.SemaphoreType.DMA,
            pltpu.SemaphoreType.DMA,
        ],
    )
    return run(user_indices.astype(jnp.int32),
               item_indices.astype(jnp.int32),
               user_emb, item_emb)
